# Initial kernel scaffold; baseline (speedup 1.0000x reference)
import jax
import jax.numpy as jnp
from jax.experimental import pallas as pl


def kernel(inputs):
    raise NotImplementedError("write your pallas kernel here")


# speedup vs baseline: 60.1286x; 60.1286x over previous
import functools

import jax
import jax.numpy as jnp
from jax import lax
from jax.experimental import pallas as pl
from jax.experimental.pallas import tpu as pltpu
from jax.experimental.pallas import tpu_sc as plsc

K_POOL = 3
BATCH, SEQ, CH = 4, 8192, 768
NUM_CORES, NUM_SUBCORES, LANES = 2, 16, 16
NUM_WORKERS = NUM_CORES * NUM_SUBCORES
SLICES = NUM_WORKERS // BATCH
ROWS_PER_WORKER = SEQ // SLICES
CHUNK = 64
NCHUNK = ROWS_PER_WORKER // CHUNK
GROUPS = CH // LANES
QUAD = 4
NEG_INF = float("-inf")


def _insert(v, m1, m2, m3):
    b1 = v > m1
    b2 = v > m2
    b3 = v > m3
    nm1 = jnp.where(b1, v, m1)
    nm2 = jnp.where(b1, m1, jnp.where(b2, v, m2))
    nm3 = jnp.where(b2, m2, jnp.where(b3, v, m3))
    return nm1, nm2, nm3


def _phase1_body(x_hbm, part_hbm, buf0, buf1, acc, sem0, sem1):
    cid = lax.axis_index("c")
    sid = lax.axis_index("s")
    wid = sid * NUM_CORES + cid
    b = wid // SLICES
    sl = wid % SLICES
    r0 = sl * ROWS_PER_WORKER

    def init_acc(g, _):
        neg = jnp.full((LANES,), NEG_INF, jnp.float32)
        for j in range(K_POOL):
            acc[j, pl.ds(g * LANES, LANES)] = neg
        return 0

    lax.fori_loop(0, GROUPS, init_acc, 0)

    def chunk_src(idx):
        return x_hbm.at[b, pl.ds(r0 + idx * CHUNK, CHUNK), :]

    pltpu.async_copy(chunk_src(0), buf0, sem0)
    pltpu.async_copy(chunk_src(1), buf1, sem1)

    def compute(buf):
        for q in range(GROUPS // QUAD):
            cols = [(q * QUAD + j) * LANES for j in range(QUAD)]
            init = []
            for c0 in cols:
                init += [acc[j, pl.ds(c0, LANES)] for j in range(K_POOL)]

            def row_body(r, carry, cols=cols):
                out = []
                for j, c0 in enumerate(cols):
                    v = buf[r, pl.ds(c0, LANES)]
                    out += list(_insert(v, *carry[3 * j:3 * j + 3]))
                return tuple(out)

            res = lax.fori_loop(0, CHUNK, row_body, tuple(init))
            for j, c0 in enumerate(cols):
                for k in range(K_POOL):
                    acc[k, pl.ds(c0, LANES)] = res[3 * j + k]

    def outer(i, _):
        for ph, (buf, sem) in enumerate(((buf0, sem0), (buf1, sem1))):
            idx = i * 2 + ph
            pltpu.make_async_copy(chunk_src(0), buf, sem).wait()
            compute(buf)

            @pl.when(i < NCHUNK // 2 - 1)
            def _():
                pltpu.async_copy(chunk_src(idx + 2), buf, sem)

        return 0

    lax.fori_loop(0, NCHUNK // 2, outer, 0)
    pltpu.sync_copy(acc, part_hbm.at[b, sl])


def _merge_body(p_ref, o_ref):
    x = p_ref[0]
    m1 = jnp.max(x, axis=0, keepdims=True)
    c1 = jnp.sum((x == m1).astype(jnp.float32), axis=0, keepdims=True)
    s2 = jnp.max(jnp.where(x < m1, x, NEG_INF), axis=0, keepdims=True)
    c2 = jnp.sum((x == s2).astype(jnp.float32), axis=0, keepdims=True)
    s3 = jnp.max(jnp.where(x < s2, x, NEG_INF), axis=0, keepdims=True)
    m2 = jnp.where(c1 >= 2, m1, s2)
    m3 = jnp.where(c1 >= 3, m1,
                   jnp.where((c1 == 2) | (c2 >= 2), s2, s3))
    o_ref[0] = jnp.concatenate([m1, m2, m3], axis=0)


@jax.jit
def kernel(inputs):
    mesh = plsc.VectorSubcoreMesh(core_axis_name="c", subcore_axis_name="s")
    phase1 = functools.partial(
        pl.kernel,
        out_type=jax.ShapeDtypeStruct((BATCH, SLICES, K_POOL, CH), jnp.float32),
        mesh=mesh,
        scratch_types=[
            pltpu.VMEM((CHUNK, CH), jnp.float32),
            pltpu.VMEM((CHUNK, CH), jnp.float32),
            pltpu.VMEM((K_POOL, CH), jnp.float32),
            pltpu.SemaphoreType.DMA,
            pltpu.SemaphoreType.DMA,
        ],
    )(_phase1_body)
    partial_topk = phase1(inputs)

    cands = partial_topk.reshape(BATCH, SLICES * K_POOL, CH)
    merged = pl.pallas_call(
        _merge_body,
        grid=(BATCH,),
        in_specs=[pl.BlockSpec((1, SLICES * K_POOL, CH), lambda i: (i, 0, 0))],
        out_specs=pl.BlockSpec((1, K_POOL, CH), lambda i: (i, 0, 0)),
        out_shape=jax.ShapeDtypeStruct((BATCH, K_POOL, CH), jnp.float32),
    )(cands)

    return merged.transpose(0, 2, 1).reshape(BATCH, CH * K_POOL)


# speedup vs baseline: 79.3827x; 1.3202x over previous
import functools

import jax
import jax.numpy as jnp
from jax import lax
from jax.experimental import pallas as pl
from jax.experimental.pallas import tpu as pltpu
from jax.experimental.pallas import tpu_sc as plsc

K_POOL = 3
BATCH, SEQ, CH = 4, 8192, 768
NUM_CORES, NUM_SUBCORES, LANES = 2, 16, 16
NUM_WORKERS = NUM_CORES * NUM_SUBCORES
SLICES = NUM_WORKERS // BATCH
ROWS_PER_WORKER = SEQ // SLICES
CHUNK = 64
NCHUNK = ROWS_PER_WORKER // CHUNK
GROUPS = CH // LANES
QUAD = 4
NEG_INF = float("-inf")


def _quad_top3(a, b, c, d):
    h1, l1 = jnp.maximum(a, b), jnp.minimum(a, b)
    h2, l2 = jnp.maximum(c, d), jnp.minimum(c, d)
    q1, hl = jnp.maximum(h1, h2), jnp.minimum(h1, h2)
    ml = jnp.maximum(l1, l2)
    return q1, jnp.maximum(hl, ml), jnp.minimum(hl, ml)


def _merge_top3(m1, m2, m3, q1, q2, q3):
    r1 = jnp.maximum(m1, q1)
    r2 = jnp.maximum(jnp.minimum(m1, q1), jnp.maximum(m2, q2))
    r3 = jnp.maximum(jnp.maximum(m3, q3),
                     jnp.maximum(jnp.minimum(m1, q2), jnp.minimum(m2, q1)))
    return r1, r2, r3


def _phase1_body(x_hbm, part_hbm, buf0, buf1, acc, sem0, sem1):
    cid = lax.axis_index("c")
    sid = lax.axis_index("s")
    wid = sid * NUM_CORES + cid
    b = wid // SLICES
    sl = wid % SLICES
    r0 = sl * ROWS_PER_WORKER

    def init_acc(g, _):
        neg = jnp.full((LANES,), NEG_INF, jnp.float32)
        for j in range(K_POOL):
            acc[j, pl.ds(g * LANES, LANES)] = neg
        return 0

    lax.fori_loop(0, GROUPS, init_acc, 0)

    def chunk_src(idx):
        return x_hbm.at[b, pl.ds(r0 + idx * CHUNK, CHUNK), :]

    pltpu.async_copy(chunk_src(0), buf0, sem0)
    pltpu.async_copy(chunk_src(1), buf1, sem1)

    def compute(buf):
        for q in range(GROUPS // QUAD):
            cols = [(q * QUAD + j) * LANES for j in range(QUAD)]
            init = []
            for c0 in cols:
                init += [acc[j, pl.ds(c0, LANES)] for j in range(K_POOL)]

            def quad_body(t, carry, cols=cols):
                r = t * 4
                out = []
                for j, c0 in enumerate(cols):
                    vals = [buf[r + i, pl.ds(c0, LANES)] for i in range(4)]
                    q123 = _quad_top3(*vals)
                    out += list(_merge_top3(*carry[3 * j:3 * j + 3], *q123))
                return tuple(out)

            res = lax.fori_loop(0, CHUNK // 4, quad_body, tuple(init))
            for j, c0 in enumerate(cols):
                for k in range(K_POOL):
                    acc[k, pl.ds(c0, LANES)] = res[3 * j + k]

    def outer(i, _):
        for ph, (buf, sem) in enumerate(((buf0, sem0), (buf1, sem1))):
            idx = i * 2 + ph
            pltpu.make_async_copy(chunk_src(0), buf, sem).wait()
            compute(buf)

            @pl.when(i < NCHUNK // 2 - 1)
            def _():
                pltpu.async_copy(chunk_src(idx + 2), buf, sem)

        return 0

    lax.fori_loop(0, NCHUNK // 2, outer, 0)
    pltpu.sync_copy(acc, part_hbm.at[b, sl])


def _merge_body(p_ref, o_ref):
    x = p_ref[0]
    m1 = jnp.max(x, axis=0, keepdims=True)
    c1 = jnp.sum((x == m1).astype(jnp.float32), axis=0, keepdims=True)
    s2 = jnp.max(jnp.where(x < m1, x, NEG_INF), axis=0, keepdims=True)
    c2 = jnp.sum((x == s2).astype(jnp.float32), axis=0, keepdims=True)
    s3 = jnp.max(jnp.where(x < s2, x, NEG_INF), axis=0, keepdims=True)
    m2 = jnp.where(c1 >= 2, m1, s2)
    m3 = jnp.where(c1 >= 3, m1,
                   jnp.where((c1 == 2) | (c2 >= 2), s2, s3))
    o_ref[0] = jnp.concatenate([m1, m2, m3], axis=0)


@jax.jit
def kernel(inputs):
    mesh = plsc.VectorSubcoreMesh(core_axis_name="c", subcore_axis_name="s")
    phase1 = functools.partial(
        pl.kernel,
        out_type=jax.ShapeDtypeStruct((BATCH, SLICES, K_POOL, CH), jnp.float32),
        mesh=mesh,
        scratch_types=[
            pltpu.VMEM((CHUNK, CH), jnp.float32),
            pltpu.VMEM((CHUNK, CH), jnp.float32),
            pltpu.VMEM((K_POOL, CH), jnp.float32),
            pltpu.SemaphoreType.DMA,
            pltpu.SemaphoreType.DMA,
        ],
    )(_phase1_body)
    partial_topk = phase1(inputs)

    cands = partial_topk.reshape(BATCH, SLICES * K_POOL, CH)
    merged = pl.pallas_call(
        _merge_body,
        grid=(BATCH,),
        in_specs=[pl.BlockSpec((1, SLICES * K_POOL, CH), lambda i: (i, 0, 0))],
        out_specs=pl.BlockSpec((1, K_POOL, CH), lambda i: (i, 0, 0)),
        out_shape=jax.ShapeDtypeStruct((BATCH, K_POOL, CH), jnp.float32),
    )(cands)

    return merged.transpose(0, 2, 1).reshape(BATCH, CH * K_POOL)
